# baseline (speedup 1.0000x reference)
import jax
import jax.numpy as jnp
from jax import lax
from jax.experimental import pallas as pl
from jax.experimental.pallas import tpu as pltpu
from jax.experimental.pallas import tpu_sc as plsc

M = 10000
K = 32
D = 128
DW = D // 2
N_EDGES = M * K
EPS = 1e-5


def _pq_body(h_ref, w1_ref, p_ref, q_ref):
    wb = w1_ref[D:, :]
    wp = w1_ref[:D, :] - wb
    x = h_ref[...]
    p_ref[...] = jnp.dot(x, wp, preferred_element_type=jnp.float32)
    q_ref[...] = jnp.dot(x, wb, preferred_element_type=jnp.float32
                         ).astype(jnp.bfloat16)


def _pq(h, w1):
    return pl.pallas_call(
        _pq_body,
        out_shape=[
            jax.ShapeDtypeStruct((M, D), jnp.float32),
            jax.ShapeDtypeStruct((M, D), jnp.bfloat16),
        ],
    )(h, w1)


_IDXW = 80
_SPG = 5
_GROUP = _IDXW * _SPG
_NW = 32
_PER_W = N_EDGES // _NW
_IDX_ROWS_W = _PER_W // _IDXW
_NGROUP = _PER_W // _GROUP


def _gather_body(adj_hbm, q_hbm, out_hbm, idx_v, rows_v, sem_g, sem_o):
    wid = lax.axis_index("s") * 2 + lax.axis_index("c")
    base = wid * _PER_W
    pltpu.sync_copy(adj_hbm.at[wid], idx_v)

    @pl.loop(0, _NGROUP)
    def group(g):
        b = lax.rem(g, 2)
        off = base + g * _GROUP

        @pl.when(g >= 2)
        def _():
            off2 = base + (g - 2) * _GROUP
            pltpu.make_async_copy(
                rows_v.at[b], out_hbm.at[pl.ds(off2, _GROUP)], sem_o
            ).wait()

        handles = [
            pltpu.async_copy(
                q_hbm.at[idx_v.at[g * _SPG + k]],
                rows_v.at[b, pl.ds(k * _IDXW, _IDXW)],
                sem_g,
            )
            for k in range(_SPG)
        ]
        for hnd in handles:
            hnd.wait()

        pltpu.async_copy(rows_v.at[b], out_hbm.at[pl.ds(off, _GROUP)], sem_o)

    for gg in (_NGROUP - 2, _NGROUP - 1):
        pltpu.make_async_copy(
            rows_v.at[gg % 2],
            out_hbm.at[pl.ds(base + gg * _GROUP, _GROUP)],
            sem_o,
        ).wait()


def _gather_sc(q, adj_rows):
    mesh = plsc.VectorSubcoreMesh(core_axis_name="c", subcore_axis_name="s")
    return pl.kernel(
        _gather_body,
        out_type=jax.ShapeDtypeStruct((N_EDGES, DW), jnp.int32),
        mesh=mesh,
        compiler_params=pltpu.CompilerParams(use_tc_tiling_on_sc=False),
        scratch_types=[
            pltpu.VMEM((_IDX_ROWS_W, _IDXW), jnp.int32),
            pltpu.VMEM((2, _GROUP, DW), jnp.int32),
            pltpu.SemaphoreType.DMA,
            pltpu.SemaphoreType.DMA,
        ],
    )(adj_rows, q)


_TN = 1000
_GRID = M // _TN


def _x1_body(p_ref, qg_ref, acc_ref):
    x1 = p_ref[...][:, None, :] + qg_ref[...].astype(jnp.float32)
    s = jnp.sum(x1, axis=(0, 1))[None, :]
    ss = jnp.sum(x1 * x1, axis=(0, 1))[None, :]

    @pl.when(pl.program_id(0) == 0)
    def _():
        acc_ref[...] = jnp.zeros_like(acc_ref)

    acc_ref[...] += jnp.concatenate([s, ss], axis=0)


def _x1_pass(p, qg3):
    return pl.pallas_call(
        _x1_body,
        grid=(_GRID,),
        in_specs=[
            pl.BlockSpec((_TN, D), lambda i: (i, 0)),
            pl.BlockSpec((_TN, K, D), lambda i: (i, 0, 0)),
        ],
        out_specs=pl.BlockSpec((2, D), lambda i: (0, 0)),
        out_shape=jax.ShapeDtypeStruct((2, D), jnp.float32),
    )(p, qg3)


def _bn_coeffs(sums_ref, gamma_ref, beta_ref):
    mean = sums_ref[0:1, :] * (1.0 / N_EDGES)
    ex2 = sums_ref[1:2, :] * (1.0 / N_EDGES)
    var = ex2 - mean * mean
    inv = lax.rsqrt(var + EPS)
    scale = gamma_ref[...] * inv
    shift = beta_ref[...] - mean * scale
    return scale, shift


def _main_body(p_ref, qg_ref, sums1_ref, g1_ref, b1_ref, w2_ref,
               maxed_ref, acc2_ref):
    scale1, shift1 = _bn_coeffs(sums1_ref, g1_ref, b1_ref)
    x1 = p_ref[...][:, None, :] + qg_ref[...].astype(jnp.float32)
    y = jnp.maximum(x1 * scale1[None, :, :] + shift1[None, :, :], 0.0)
    y2 = y.reshape(_TN * K, D).astype(jnp.bfloat16)
    x2 = jnp.dot(y2, w2_ref[...].astype(jnp.bfloat16),
                 preferred_element_type=jnp.float32)
    s = jnp.sum(x2, axis=0)[None, :]
    ss = jnp.sum(x2 * x2, axis=0)[None, :]

    @pl.when(pl.program_id(0) == 0)
    def _():
        acc2_ref[...] = jnp.zeros_like(acc2_ref)

    acc2_ref[...] += jnp.concatenate([s, ss], axis=0)
    maxed_ref[...] = jnp.max(x2.reshape(_TN, K, D), axis=1)


def _main(p, qg3, sums1, gamma1, beta1, w2):
    return pl.pallas_call(
        _main_body,
        grid=(_GRID,),
        in_specs=[
            pl.BlockSpec((_TN, D), lambda i: (i, 0)),
            pl.BlockSpec((_TN, K, D), lambda i: (i, 0, 0)),
            pl.BlockSpec((2, D), lambda i: (0, 0)),
            pl.BlockSpec((1, D), lambda i: (0, 0)),
            pl.BlockSpec((1, D), lambda i: (0, 0)),
            pl.BlockSpec((D, D), lambda i: (0, 0)),
        ],
        out_specs=[
            pl.BlockSpec((_TN, D), lambda i: (i, 0)),
            pl.BlockSpec((2, D), lambda i: (0, 0)),
        ],
        out_shape=[
            jax.ShapeDtypeStruct((M, D), jnp.float32),
            jax.ShapeDtypeStruct((2, D), jnp.float32),
        ],
    )(p, qg3, sums1, gamma1, beta1, w2)


def _final_body(maxed_ref, sums2_ref, g2_ref, b2_ref, out_ref):
    scale2, shift2 = _bn_coeffs(sums2_ref, g2_ref, b2_ref)
    out_ref[...] = jnp.maximum(maxed_ref[...] * scale2 + shift2, 0.0)


def _final(maxed, sums2, gamma2, beta2):
    return pl.pallas_call(
        _final_body,
        out_shape=jax.ShapeDtypeStruct((M, D), jnp.float32),
    )(maxed, sums2, gamma2, beta2)


def kernel(h, adj, W1, gamma1, beta1, W2, gamma2, beta2):
    adj_rows = adj.astype(jnp.int32).reshape(_NW, _IDX_ROWS_W, _IDXW)
    p, qb = _pq(h, W1)
    q_packed = lax.bitcast_convert_type(qb.reshape(M, DW, 2), jnp.int32)
    qg = _gather_sc(q_packed, adj_rows)
    qg3 = lax.bitcast_convert_type(qg, jnp.bfloat16).reshape(M, K, D)
    sums1 = _x1_pass(p, qg3)
    maxed, sums2 = _main(p, qg3, sums1,
                         gamma1.reshape(1, D), beta1.reshape(1, D), W2)
    return _final(maxed, sums2, gamma2.reshape(1, D), beta2.reshape(1, D))


# speedup vs baseline: 1.0960x; 1.0960x over previous
import jax
import jax.numpy as jnp
from jax import lax
from jax.experimental import pallas as pl
from jax.experimental.pallas import tpu as pltpu
from jax.experimental.pallas import tpu_sc as plsc

M = 10000
K = 32
D = 128
DH = D // 2
N_EDGES = M * K
EPS = 1e-5


def _pq_body(h_ref, w1_ref, p_ref, q_ref):
    wb = w1_ref[D:, :]
    wp = w1_ref[:D, :] - wb
    x = h_ref[...]
    p = jnp.dot(x, wp, preferred_element_type=jnp.float32)
    q = jnp.dot(x, wb, preferred_element_type=jnp.float32)
    p_ref[...] = jnp.concatenate(
        [p[:, None, :DH], p[:, None, DH:]], axis=1)
    qsplit = jnp.concatenate(
        [q[:, None, :DH], q[:, None, DH:]], axis=1).astype(jnp.bfloat16)
    q_ref[...] = pltpu.bitcast(qsplit.reshape(2 * M, DH), jnp.int32)


def _pq(h, w1):
    return pl.pallas_call(
        _pq_body,
        out_shape=[
            jax.ShapeDtypeStruct((M, 2, DH), jnp.float32),
            jax.ShapeDtypeStruct((M, DH), jnp.int32),
        ],
    )(h, w1)


_IDXW = 80
_SPG = 5
_GROUP = _IDXW * _SPG
_NW = 32
_PER_W = N_EDGES // _NW
_IDX_ROWS_W = _PER_W // _IDXW
_NGROUP = _PER_W // _GROUP


def _gather_body(adj_hbm, q_hbm, out_hbm, idx_v, rows_v, sem_g, sem_o):
    wid = lax.axis_index("s") * 2 + lax.axis_index("c")
    base = wid * _PER_W
    pltpu.sync_copy(adj_hbm.at[wid], idx_v)

    @pl.loop(0, _NGROUP)
    def group(g):
        b = lax.rem(g, 2)
        off = base + g * _GROUP

        @pl.when(g >= 2)
        def _():
            off2 = base + (g - 2) * _GROUP
            pltpu.make_async_copy(
                rows_v.at[b], out_hbm.at[pl.ds(off2, _GROUP)], sem_o
            ).wait()

        handles = [
            pltpu.async_copy(
                q_hbm.at[idx_v.at[g * _SPG + k]],
                rows_v.at[b, pl.ds(k * _IDXW, _IDXW)],
                sem_g,
            )
            for k in range(_SPG)
        ]
        for hnd in handles:
            hnd.wait()

        pltpu.async_copy(rows_v.at[b], out_hbm.at[pl.ds(off, _GROUP)], sem_o)

    for gg in (_NGROUP - 2, _NGROUP - 1):
        pltpu.make_async_copy(
            rows_v.at[gg % 2],
            out_hbm.at[pl.ds(base + gg * _GROUP, _GROUP)],
            sem_o,
        ).wait()


def _gather_sc(q_packed, adj_rows):
    mesh = plsc.VectorSubcoreMesh(core_axis_name="c", subcore_axis_name="s")
    return pl.kernel(
        _gather_body,
        out_type=jax.ShapeDtypeStruct((N_EDGES, DH), jnp.int32),
        mesh=mesh,
        compiler_params=pltpu.CompilerParams(use_tc_tiling_on_sc=False),
        scratch_types=[
            pltpu.VMEM((_IDX_ROWS_W, _IDXW), jnp.int32),
            pltpu.VMEM((2, _GROUP, DH), jnp.int32),
            pltpu.SemaphoreType.DMA,
            pltpu.SemaphoreType.DMA,
        ],
    )(adj_rows, q_packed)


_TN = 1000
_GRID = M // _TN


def _unpack(qg_words):
    bf = pltpu.bitcast(qg_words, jnp.bfloat16)
    return bf.reshape(_TN, K, 2, DH).astype(jnp.float32)


def _x1_body(p_ref, qg_ref, acc_ref):
    qg = _unpack(qg_ref[...].reshape(_TN * K, DH))
    x1 = p_ref[...][:, None, :, :] + qg
    s = jnp.sum(x1, axis=(0, 1))
    ss = jnp.sum(x1 * x1, axis=(0, 1))

    @pl.when(pl.program_id(0) == 0)
    def _():
        acc_ref[...] = jnp.zeros_like(acc_ref)

    acc_ref[...] += jnp.concatenate([s, ss], axis=0)


def _x1_pass(p, qg3):
    return pl.pallas_call(
        _x1_body,
        grid=(_GRID,),
        in_specs=[
            pl.BlockSpec((_TN, 2, DH), lambda i: (i, 0, 0)),
            pl.BlockSpec((_TN, K, DH), lambda i: (i, 0, 0)),
        ],
        out_specs=pl.BlockSpec((4, DH), lambda i: (0, 0)),
        out_shape=jax.ShapeDtypeStruct((4, DH), jnp.float32),
    )(p, qg3)


def _bn1_coeffs(sums_ref, g1_ref, b1_ref):
    mean = sums_ref[0:2, :] * (1.0 / N_EDGES)
    ex2 = sums_ref[2:4, :] * (1.0 / N_EDGES)
    var = ex2 - mean * mean
    inv = lax.rsqrt(var + EPS)
    scale = g1_ref[...] * inv
    shift = b1_ref[...] - mean * scale
    return scale, shift


def _bn_coeffs(sums_ref, gamma_ref, beta_ref):
    mean = sums_ref[0:1, :] * (1.0 / N_EDGES)
    ex2 = sums_ref[1:2, :] * (1.0 / N_EDGES)
    var = ex2 - mean * mean
    inv = lax.rsqrt(var + EPS)
    scale = gamma_ref[...] * inv
    shift = beta_ref[...] - mean * scale
    return scale, shift


def _main_body(p_ref, qg_ref, sums1_ref, g1_ref, b1_ref, w2_ref,
               maxed_ref, acc2_ref):
    scale1, shift1 = _bn1_coeffs(sums1_ref, g1_ref, b1_ref)
    qg = _unpack(qg_ref[...].reshape(_TN * K, DH))
    x1 = p_ref[...][:, None, :, :] + qg
    y = jnp.maximum(x1 * scale1[None, None, :, :]
                    + shift1[None, None, :, :], 0.0)
    ya = y[:, :, 0, :].reshape(_TN * K, DH).astype(jnp.bfloat16)
    yb = y[:, :, 1, :].reshape(_TN * K, DH).astype(jnp.bfloat16)
    w2 = w2_ref[...].astype(jnp.bfloat16)
    x2 = (jnp.dot(ya, w2[:DH, :], preferred_element_type=jnp.float32)
          + jnp.dot(yb, w2[DH:, :], preferred_element_type=jnp.float32))
    s = jnp.sum(x2, axis=0)[None, :]
    ss = jnp.sum(x2 * x2, axis=0)[None, :]

    @pl.when(pl.program_id(0) == 0)
    def _():
        acc2_ref[...] = jnp.zeros_like(acc2_ref)

    acc2_ref[...] += jnp.concatenate([s, ss], axis=0)
    maxed_ref[...] = jnp.max(x2.reshape(_TN, K, D), axis=1)


def _main(p, qg3, sums1, gamma1, beta1, w2):
    return pl.pallas_call(
        _main_body,
        grid=(_GRID,),
        in_specs=[
            pl.BlockSpec((_TN, 2, DH), lambda i: (i, 0, 0)),
            pl.BlockSpec((_TN, K, DH), lambda i: (i, 0, 0)),
            pl.BlockSpec((4, DH), lambda i: (0, 0)),
            pl.BlockSpec((2, DH), lambda i: (0, 0)),
            pl.BlockSpec((2, DH), lambda i: (0, 0)),
            pl.BlockSpec((D, D), lambda i: (0, 0)),
        ],
        out_specs=[
            pl.BlockSpec((_TN, D), lambda i: (i, 0)),
            pl.BlockSpec((2, D), lambda i: (0, 0)),
        ],
        out_shape=[
            jax.ShapeDtypeStruct((M, D), jnp.float32),
            jax.ShapeDtypeStruct((2, D), jnp.float32),
        ],
    )(p, qg3, sums1, gamma1, beta1, w2)


def _final_body(maxed_ref, sums2_ref, g2_ref, b2_ref, out_ref):
    scale2, shift2 = _bn_coeffs(sums2_ref, g2_ref, b2_ref)
    out_ref[...] = jnp.maximum(maxed_ref[...] * scale2 + shift2, 0.0)


def _final(maxed, sums2, gamma2, beta2):
    return pl.pallas_call(
        _final_body,
        out_shape=jax.ShapeDtypeStruct((M, D), jnp.float32),
    )(maxed, sums2, gamma2, beta2)


def kernel(h, adj, W1, gamma1, beta1, W2, gamma2, beta2):
    adj_rows = adj.astype(jnp.int32).reshape(_NW, _IDX_ROWS_W, _IDXW)
    p, q_packed = _pq(h, W1)
    qg = _gather_sc(q_packed, adj_rows)
    qg3 = qg.reshape(M, K, DH)
    sums1 = _x1_pass(p, qg3)
    maxed, sums2 = _main(p, qg3, sums1,
                         gamma1.reshape(2, DH), beta1.reshape(2, DH), W2)
    return _final(maxed, sums2, gamma2.reshape(1, D), beta2.reshape(1, D))


# speedup vs baseline: 5.3992x; 4.9263x over previous
import jax
import jax.numpy as jnp
from jax import lax
from jax.experimental import pallas as pl
from jax.experimental.pallas import tpu as pltpu
from jax.experimental.pallas import tpu_sc as plsc

M = 10000
K = 32
D = 128
N_EDGES = M * K
EPS = 1e-5


def _pq_body(h_ref, w1_ref, p_ref, q_ref):
    wb = w1_ref[D:, :]
    wp = w1_ref[:D, :] - wb
    x = h_ref[...]
    p_ref[...] = jnp.dot(x, wp, preferred_element_type=jnp.float32)
    q_ref[...] = jnp.dot(x, wb, preferred_element_type=jnp.float32)


def _pq(h, w1):
    return pl.pallas_call(
        _pq_body,
        out_shape=[
            jax.ShapeDtypeStruct((M, D), jnp.float32),
            jax.ShapeDtypeStruct((M, D), jnp.float32),
        ],
    )(h, w1)


_IDXW = 80
_SPG = 5
_GROUP = _IDXW * _SPG
_NW = 32
_PER_W = N_EDGES // _NW
_IDX_ROWS_W = _PER_W // _IDXW
_NGROUP = _PER_W // _GROUP


def _gather_body(adj_hbm, q_hbm, out_hbm, idx_v, rows_v, sem_g, sem_o):
    wid = lax.axis_index("s") * 2 + lax.axis_index("c")
    base = wid * _PER_W
    pltpu.sync_copy(adj_hbm.at[wid], idx_v)

    @pl.loop(0, _NGROUP)
    def group(g):
        b = lax.rem(g, 2)
        off = base + g * _GROUP

        @pl.when(g >= 2)
        def _():
            off2 = base + (g - 2) * _GROUP
            pltpu.make_async_copy(
                rows_v.at[b], out_hbm.at[pl.ds(off2, _GROUP)], sem_o
            ).wait()

        handles = [
            pltpu.async_copy(
                q_hbm.at[idx_v.at[g * _SPG + k]],
                rows_v.at[b, pl.ds(k * _IDXW, _IDXW)],
                sem_g,
            )
            for k in range(_SPG)
        ]
        for hnd in handles:
            hnd.wait()

        pltpu.async_copy(rows_v.at[b], out_hbm.at[pl.ds(off, _GROUP)], sem_o)

    for gg in (_NGROUP - 2, _NGROUP - 1):
        pltpu.make_async_copy(
            rows_v.at[gg % 2],
            out_hbm.at[pl.ds(base + gg * _GROUP, _GROUP)],
            sem_o,
        ).wait()


def _gather_sc(q, adj_rows):
    mesh = plsc.VectorSubcoreMesh(core_axis_name="c", subcore_axis_name="s")
    return pl.kernel(
        _gather_body,
        out_type=jax.ShapeDtypeStruct((N_EDGES, D), jnp.float32),
        mesh=mesh,
        scratch_types=[
            pltpu.VMEM((_IDX_ROWS_W, _IDXW), jnp.int32),
            pltpu.VMEM((2, _GROUP, D), jnp.float32),
            pltpu.SemaphoreType.DMA,
            pltpu.SemaphoreType.DMA,
        ],
    )(adj_rows, q)


_TN = 1000
_GRID = M // _TN


def _x1_body(p_ref, qg_ref, acc_ref):
    x1 = p_ref[...][:, None, :] + qg_ref[...]
    s = jnp.sum(x1, axis=(0, 1))[None, :]
    ss = jnp.sum(x1 * x1, axis=(0, 1))[None, :]

    @pl.when(pl.program_id(0) == 0)
    def _():
        acc_ref[...] = jnp.zeros_like(acc_ref)

    acc_ref[...] += jnp.concatenate([s, ss], axis=0)


def _x1_pass(p, qg3):
    return pl.pallas_call(
        _x1_body,
        grid=(_GRID,),
        in_specs=[
            pl.BlockSpec((_TN, D), lambda i: (i, 0)),
            pl.BlockSpec((_TN, K, D), lambda i: (i, 0, 0)),
        ],
        out_specs=pl.BlockSpec((2, D), lambda i: (0, 0)),
        out_shape=jax.ShapeDtypeStruct((2, D), jnp.float32),
    )(p, qg3)


def _bn_coeffs(sums_ref, gamma_ref, beta_ref):
    mean = sums_ref[0:1, :] * (1.0 / N_EDGES)
    ex2 = sums_ref[1:2, :] * (1.0 / N_EDGES)
    var = ex2 - mean * mean
    inv = lax.rsqrt(var + EPS)
    scale = gamma_ref[...] * inv
    shift = beta_ref[...] - mean * scale
    return scale, shift


def _main_body(p_ref, qg_ref, sums1_ref, g1_ref, b1_ref, w2_ref,
               maxed_ref, acc2_ref):
    scale1, shift1 = _bn_coeffs(sums1_ref, g1_ref, b1_ref)
    x1 = p_ref[...][:, None, :] + qg_ref[...]
    y = jnp.maximum(x1 * scale1[None, :, :] + shift1[None, :, :], 0.0)
    y2 = y.reshape(_TN * K, D).astype(jnp.bfloat16)
    x2 = jnp.dot(y2, w2_ref[...].astype(jnp.bfloat16),
                 preferred_element_type=jnp.float32)
    s = jnp.sum(x2, axis=0)[None, :]
    ss = jnp.sum(x2 * x2, axis=0)[None, :]

    @pl.when(pl.program_id(0) == 0)
    def _():
        acc2_ref[...] = jnp.zeros_like(acc2_ref)

    acc2_ref[...] += jnp.concatenate([s, ss], axis=0)
    maxed_ref[...] = jnp.max(x2.reshape(_TN, K, D), axis=1)


def _main(p, qg3, sums1, gamma1, beta1, w2):
    return pl.pallas_call(
        _main_body,
        grid=(_GRID,),
        in_specs=[
            pl.BlockSpec((_TN, D), lambda i: (i, 0)),
            pl.BlockSpec((_TN, K, D), lambda i: (i, 0, 0)),
            pl.BlockSpec((2, D), lambda i: (0, 0)),
            pl.BlockSpec((1, D), lambda i: (0, 0)),
            pl.BlockSpec((1, D), lambda i: (0, 0)),
            pl.BlockSpec((D, D), lambda i: (0, 0)),
        ],
        out_specs=[
            pl.BlockSpec((_TN, D), lambda i: (i, 0)),
            pl.BlockSpec((2, D), lambda i: (0, 0)),
        ],
        out_shape=[
            jax.ShapeDtypeStruct((M, D), jnp.float32),
            jax.ShapeDtypeStruct((2, D), jnp.float32),
        ],
    )(p, qg3, sums1, gamma1, beta1, w2)


def _final_body(maxed_ref, sums2_ref, g2_ref, b2_ref, out_ref):
    scale2, shift2 = _bn_coeffs(sums2_ref, g2_ref, b2_ref)
    out_ref[...] = jnp.maximum(maxed_ref[...] * scale2 + shift2, 0.0)


def _final(maxed, sums2, gamma2, beta2):
    return pl.pallas_call(
        _final_body,
        out_shape=jax.ShapeDtypeStruct((M, D), jnp.float32),
    )(maxed, sums2, gamma2, beta2)


def kernel(h, adj, W1, gamma1, beta1, W2, gamma2, beta2):
    adj_rows = adj.astype(jnp.int32).reshape(_NW, _IDX_ROWS_W, _IDXW)
    p, q = _pq(h, W1)
    qg = _gather_sc(q, adj_rows)
    qg3 = qg.reshape(M, K, D)
    sums1 = _x1_pass(p, qg3)
    maxed, sums2 = _main(p, qg3, sums1,
                         gamma1.reshape(1, D), beta1.reshape(1, D), W2)
    return _final(maxed, sums2, gamma2.reshape(1, D), beta2.reshape(1, D))
